# baseline (speedup 1.0000x reference)
import functools

import jax
import jax.numpy as jnp
from jax import lax
from jax.experimental import pallas as pl
from jax.experimental.pallas import tpu as pltpu
from jax.experimental.pallas import tpu_sc as plsc

_K = 512
_BINS = _K + 1
_BS = 4
_BW = 528
_L = 16


def _codes_body(x_ref, c_ref, o_ref):
    xf = x_ref[...]
    cm = c_ref[...]
    cn = jnp.sum(cm * cm, axis=0, keepdims=True)
    prod = lax.dot_general(
        xf, cm, (((1,), (0,)), ((), ())),
        precision=lax.Precision.HIGHEST,
        preferred_element_type=jnp.float32,
    )
    s = cn - 2.0 * prod
    m = jnp.min(s, axis=1, keepdims=True)
    ki = lax.broadcasted_iota(jnp.int32, s.shape, 1)
    o_ref[...] = jnp.min(jnp.where(s == m, ki, _K), axis=1, keepdims=True) + 1


def _hist_body(codes_hbm, zeros_hbm, out_hbm, codes_v, buf):
    wid = lax.axis_index("s") * 2 + lax.axis_index("c")

    @pl.when(wid < 4)
    def _():
        g = wid
        pltpu.sync_copy(zeros_hbm, buf)
        pltpu.sync_copy(codes_hbm.at[pl.ds(g * 256, 256)], codes_v)
        rowbase = lax.iota(jnp.int32, _L) * _BW
        ones = jnp.full((_L,), 1.0 / 16.0, dtype=jnp.float32)
        for p in range(16):
            cv = codes_v[pl.ds(p * _L, _L)]
            plsc.addupdate_scatter(buf, [rowbase + cv], ones)
        pltpu.sync_copy(buf, out_hbm.at[g])


def kernel(x, cluster_centers):
    B, C, H, W = x.shape
    nh, nw = H // _BS, W // _BS
    xb = (
        x.transpose(0, 2, 3, 1)
        .reshape(B, nh, _BS, nw, _BS, C)
        .transpose(0, 2, 4, 1, 3, 5)
        .reshape(B * H * W, C)
    )
    cm = cluster_centers.reshape(_K, C).T
    codes = pl.pallas_call(
        _codes_body,
        out_shape=jax.ShapeDtypeStruct((B * H * W, 1), jnp.int32),
    )(xb, cm).reshape(B * H * W)

    zeros = jnp.zeros((_L * _BW,), dtype=jnp.float32)
    hist = pl.kernel(
        _hist_body,
        out_type=jax.ShapeDtypeStruct((B, _L * _BW), jnp.float32),
        mesh=plsc.VectorSubcoreMesh(core_axis_name="c", subcore_axis_name="s"),
        compiler_params=pltpu.CompilerParams(needs_layout_passes=False),
        scratch_types=[
            pltpu.VMEM((256,), jnp.int32),
            pltpu.VMEM((_L * _BW,), jnp.float32),
        ],
    )(codes, zeros)
    return hist.reshape(B, _L, _BW)[:, :, :_BINS]


# speedup vs baseline: 1.0501x; 1.0501x over previous
import functools

import jax
import jax.numpy as jnp
from jax import lax
from jax.experimental import pallas as pl
from jax.experimental.pallas import tpu as pltpu
from jax.experimental.pallas import tpu_sc as plsc

_K = 512
_BINS = _K + 1
_BS = 4
_BW = 528
_L = 16


def _codes_body(x_ref, c_ref, o_ref):
    xf = x_ref[...]
    cm = c_ref[...]
    cn = jnp.sum(cm * cm, axis=0, keepdims=True)
    prod = lax.dot_general(
        xf, cm, (((1,), (0,)), ((), ())),
        precision=lax.Precision.HIGHEST,
        preferred_element_type=jnp.float32,
    )
    s = cn - 2.0 * prod
    m = jnp.min(s, axis=1, keepdims=True)
    ki = lax.broadcasted_iota(jnp.int32, s.shape, 1)
    o_ref[...] = jnp.min(jnp.where(s == m, ki, _K), axis=1, keepdims=True) + 1


def _hist_body(codes_hbm, zeros_hbm, out_hbm, codes_v, buf):
    wid = lax.axis_index("s")

    @pl.when(wid < 4)
    def _():
        g = wid
        pltpu.sync_copy(zeros_hbm, buf)
        pltpu.sync_copy(codes_hbm.at[pl.ds(g * 256, 256)], codes_v)
        rowbase = lax.iota(jnp.int32, _L) * _BW
        ones = jnp.full((_L,), 1.0 / 16.0, dtype=jnp.float32)
        for p in range(16):
            cv = codes_v[pl.ds(p * _L, _L)]
            plsc.addupdate_scatter(buf, [rowbase + cv], ones)
        pltpu.sync_copy(buf, out_hbm.at[g])


def kernel(x, cluster_centers):
    B, C, H, W = x.shape
    nh, nw = H // _BS, W // _BS
    xb = (
        x.transpose(0, 2, 3, 1)
        .reshape(B, nh, _BS, nw, _BS, C)
        .transpose(0, 2, 4, 1, 3, 5)
        .reshape(B * H * W, C)
    )
    cm = cluster_centers.reshape(_K, C).T
    codes = pl.pallas_call(
        _codes_body,
        out_shape=jax.ShapeDtypeStruct((B * H * W, 1), jnp.int32),
    )(xb, cm).reshape(B * H * W)

    zeros = jnp.zeros((_L * _BW,), dtype=jnp.float32)
    hist = pl.kernel(
        _hist_body,
        out_type=jax.ShapeDtypeStruct((B, _L * _BW), jnp.float32),
        mesh=plsc.VectorSubcoreMesh(
            core_axis_name="c", subcore_axis_name="s", num_cores=1
        ),
        compiler_params=pltpu.CompilerParams(needs_layout_passes=False),
        scratch_types=[
            pltpu.VMEM((256,), jnp.int32),
            pltpu.VMEM((_L * _BW,), jnp.float32),
        ],
    )(codes, zeros)
    return hist.reshape(B, _L, _BW)[:, :, :_BINS]


# speedup vs baseline: 1.0510x; 1.0008x over previous
import jax
import jax.numpy as jnp
from jax import lax
from jax.experimental import pallas as pl
from jax.experimental.pallas import tpu as pltpu
from jax.experimental.pallas import tpu_sc as plsc

_K = 512
_BINS = _K + 1
_BS = 4
_L = 16


def _codes_body(x_ref, c_ref, o_ref):
    cm = c_ref[...]
    cn = jnp.sum(cm * cm, axis=1, keepdims=True)
    for b in range(4):
        xb = x_ref[pl.ds(b * 96, 96), :]
        prod = lax.dot_general(
            cm, xb, (((1,), (0,)), ((), ())),
            precision=lax.Precision.HIGHEST,
            preferred_element_type=jnp.float32,
        )
        s = cn - 2.0 * prod
        m = jnp.min(s, axis=0, keepdims=True)
        ki = lax.broadcasted_iota(jnp.int32, s.shape, 0)
        code = jnp.min(jnp.where(s == m, ki, _K), axis=0, keepdims=True)
        o_ref[pl.ds(b, 1), :] = code + 1


def _hist_body(codes_hbm, zeros_hbm, out_hbm, codes_v, buf):
    wid = lax.axis_index("s")

    @pl.when(wid < 4)
    def _():
        g = wid
        pltpu.sync_copy(zeros_hbm, buf)
        pltpu.sync_copy(codes_hbm.at[pl.ds(g * 256, 256)], codes_v)
        l = lax.iota(jnp.int32, _L)
        lq, lr = l // _BS, l % _BS
        ones = jnp.full((_L,), 1.0 / 16.0, dtype=jnp.float32)
        for p in range(16):
            src = (lq * _BS + p // _BS) * 16 + lr * _BS + (p % _BS)
            cv = plsc.load_gather(codes_v, [src])
            plsc.addupdate_scatter(buf, [l * _BINS + cv], ones)
        pltpu.sync_copy(buf, out_hbm.at[g])


def kernel(x, cluster_centers):
    B, C, H, W = x.shape
    xr = x.reshape(B * C, H * W)
    cm = cluster_centers.reshape(_K, C)
    codes = pl.pallas_call(
        _codes_body,
        out_shape=jax.ShapeDtypeStruct((B, H * W), jnp.int32),
    )(xr, cm).reshape(B * H * W)

    zeros = jnp.zeros((_L * _BINS,), dtype=jnp.float32)
    hist = pl.kernel(
        _hist_body,
        out_type=jax.ShapeDtypeStruct((B, _L * _BINS), jnp.float32),
        mesh=plsc.VectorSubcoreMesh(
            core_axis_name="c", subcore_axis_name="s", num_cores=1
        ),
        compiler_params=pltpu.CompilerParams(needs_layout_passes=False),
        scratch_types=[
            pltpu.VMEM((256,), jnp.int32),
            pltpu.VMEM((_L * _BINS,), jnp.float32),
        ],
    )(codes, zeros)
    return hist.reshape(B, _L, _BINS)


# speedup vs baseline: 3.5897x; 3.4155x over previous
import jax
import jax.numpy as jnp
from jax import lax
from jax.experimental import pallas as pl

_K = 512
_BINS = _K + 1


def _body(x_ref, c_ref, o_ref):
    cm = c_ref[...]
    cn = jnp.sum(cm * cm, axis=1, keepdims=True)
    for b in range(4):
        xb = x_ref[pl.ds(b * 96, 96), :]
        prod = lax.dot_general(
            cm, xb, (((1,), (0,)), ((), ())),
            precision=lax.Precision.HIGHEST,
            preferred_element_type=jnp.float32,
        )
        s = cn - 2.0 * prod
        m = jnp.min(s, axis=0, keepdims=True)
        ki = lax.broadcasted_iota(jnp.int32, s.shape, 0)
        code = jnp.min(jnp.where(s == m, ki, _K), axis=0, keepdims=True).T
        bins = lax.broadcasted_iota(jnp.int32, (256, _K), 1)
        oh = (bins == code).astype(jnp.float32)
        hist = oh.reshape(4, 4, 4, 4, _K).sum(axis=(1, 3)).reshape(16, _K)
        full = jnp.pad(hist * (1.0 / 16.0), ((0, 0), (1, 0)))
        o_ref[pl.ds(b * 16, 16), :] = full


def kernel(x, cluster_centers):
    B, C, H, W = x.shape
    xr = x.reshape(B * C, H * W)
    cm = cluster_centers.reshape(_K, C)
    return pl.pallas_call(
        _body,
        out_shape=jax.ShapeDtypeStruct((B * 16, _BINS), jnp.float32),
    )(xr, cm).reshape(B, 16, _BINS)
